# baseline (speedup 1.0000x reference)
import functools

import jax
import jax.numpy as jnp
from jax.experimental import pallas as pl
from jax.experimental.pallas import tpu as pltpu

TH, TW = 15, 15
PAD_HT = (TH - 1) // 2
PAD_WL = (TW - 1) // 2
MARGIN = 0.25
GAMMA = 256.0
NEG = -1e30
PEN = -1e35


def _loss_body(x_ref, m_ref, o_ref, mp_a, sp_a, mn_a, sn_a, *,
               nrows, W, r0, r1, c0, c1):
    i = pl.program_id(0)
    nsub = nrows // 8

    @pl.when(i == 0)
    def _init():
        mp_a[...] = jnp.full((8, W), NEG, jnp.float32)
        sp_a[...] = jnp.zeros((8, W), jnp.float32)
        mn_a[...] = jnp.full((8, W), NEG, jnp.float32)
        sn_a[...] = jnp.zeros((8, W), jnp.float32)

    x = x_ref[...]
    tm = m_ref[...]

    rid = jax.lax.broadcasted_iota(jnp.int32, (nrows, 1), 0) + i * nrows
    rowpen = jnp.where((rid >= r0) & (rid < r1), 0.0, PEN)

    K = GAMMA * 1.4426950408889634
    y = x * K
    lp = jnp.maximum(K * (1.0 + MARGIN) - y, 0.0) * ((1.0 - MARGIN) - x)
    ln = jnp.maximum(y + K * MARGIN, 0.0) * (x - MARGIN) + rowpen
    lP = jnp.where(tm, lp, PEN).reshape(nsub, 8, W)
    lN = jnp.where(tm, PEN, ln).reshape(nsub, 8, W)

    mp_old = mp_a[...]
    mn_old = mn_a[...]
    mp = jnp.maximum(mp_old, jnp.max(lP, axis=0))
    mn = jnp.maximum(mn_old, jnp.max(lN, axis=0))

    ep = jnp.exp2(lP - mp[None, :, :])
    en = jnp.exp2(lN - mn[None, :, :])

    mp_a[...] = mp
    sp_a[...] = sp_a[...] * jnp.exp2(mp_old - mp) + jnp.sum(ep, axis=0)
    mn_a[...] = mn
    sn_a[...] = sn_a[...] * jnp.exp2(mn_old - mn) + jnp.sum(en, axis=0)

    @pl.when(i == pl.num_programs(0) - 1)
    def _fin():
        LN2 = 0.6931471805599453
        cid = jax.lax.broadcasted_iota(jnp.int32, (1, W), 1)
        col_ok = (cid >= c0) & (cid < c1)
        mp_c = mp_a[...]
        mn_c = jnp.where(col_ok, mn_a[...], NEG)
        sn_c = jnp.where(col_ok, sn_a[...], 0.0)
        Mp = jnp.max(mp_c)
        Mn = jnp.max(mn_c)
        Sp = jnp.sum(sp_a[...] * jnp.exp2(mp_c - Mp))
        Sn = jnp.sum(sn_c * jnp.exp2(mn_c - Mn))
        z = LN2 * (Mp + Mn) + jnp.log(Sp) + jnp.log(Sn)
        o_ref[0, 0] = jnp.maximum(z, 0.0) + jnp.log1p(jnp.exp(-jnp.abs(z)))


def kernel(ypred, truthMask):
    B, H, W = ypred.shape
    mh, mw = truthMask.shape[-2], truthMask.shape[-1]
    r0 = PAD_HT - 1
    r1 = min(PAD_HT - TH + mh + 2, H)
    c0 = PAD_WL - 1
    c1 = min(PAD_WL - TW + mw + 2, W)

    x = ypred.reshape(H, W)
    tm = truthMask.reshape(H, W)

    nrows = 512 if H % 512 == 0 else H
    grid = H // nrows

    out = pl.pallas_call(
        functools.partial(
            _loss_body, nrows=nrows, W=W, r0=r0, r1=r1, c0=c0, c1=c1
        ),
        grid=(grid,),
        in_specs=[
            pl.BlockSpec((nrows, W), lambda i: (i, 0)),
            pl.BlockSpec((nrows, W), lambda i: (i, 0)),
        ],
        out_specs=pl.BlockSpec(
            (1, 1), lambda i: (0, 0), memory_space=pltpu.SMEM
        ),
        out_shape=jax.ShapeDtypeStruct((1, 1), jnp.float32),
        scratch_shapes=[
            pltpu.VMEM((8, W), jnp.float32),
            pltpu.VMEM((8, W), jnp.float32),
            pltpu.VMEM((8, W), jnp.float32),
            pltpu.VMEM((8, W), jnp.float32),
        ],
        compiler_params=pltpu.CompilerParams(
            dimension_semantics=("arbitrary",),
        ),
    )(x, tm)
    return out.reshape(B)


# speedup vs baseline: 1.0320x; 1.0320x over previous
import functools

import jax
import jax.numpy as jnp
from jax.experimental import pallas as pl
from jax.experimental.pallas import tpu as pltpu

TH, TW = 15, 15
PAD_HT = (TH - 1) // 2
PAD_WL = (TW - 1) // 2
MARGIN = 0.25
GAMMA = 256.0
NEG = -1e30
PEN = -1e35


def _loss_body(x_ref, m_ref, o_ref, mp_a, sp_a, mn_a, sn_a, *,
               nrows, W, r0, r1, c0, c1):
    i = pl.program_id(0)
    nsub = nrows // 8

    @pl.when(i == 0)
    def _init():
        mp_a[...] = jnp.full((8, W), NEG, jnp.float32)
        sp_a[...] = jnp.zeros((8, W), jnp.float32)
        mn_a[...] = jnp.full((8, W), NEG, jnp.float32)
        sn_a[...] = jnp.zeros((8, W), jnp.float32)

    x = x_ref[...]
    tm = m_ref[...]

    rid = jax.lax.broadcasted_iota(jnp.int32, (nrows, 1), 0) + i * nrows
    rowpen = jnp.where((rid >= r0) & (rid < r1), 0.0, PEN)

    K = GAMMA * 1.4426950408889634
    y = x * K
    lp = jnp.maximum(K * (1.0 + MARGIN) - y, 0.0) * ((1.0 - MARGIN) - x)
    ln = jnp.maximum(y + K * MARGIN, 0.0) * (x - MARGIN) + rowpen
    lP = jnp.where(tm, lp, PEN).reshape(nsub, 8, W)
    lN = jnp.where(tm, PEN, ln).reshape(nsub, 8, W)

    mp_old = mp_a[...]
    mn_old = mn_a[...]
    mp = jnp.maximum(mp_old, jnp.max(lP, axis=0))
    mn = jnp.maximum(mn_old, jnp.max(lN, axis=0))

    ep = jnp.exp2(lP - mp[None, :, :])
    en = jnp.exp2(lN - mn[None, :, :])

    mp_a[...] = mp
    sp_a[...] = sp_a[...] * jnp.exp2(mp_old - mp) + jnp.sum(ep, axis=0)
    mn_a[...] = mn
    sn_a[...] = sn_a[...] * jnp.exp2(mn_old - mn) + jnp.sum(en, axis=0)

    @pl.when(i == pl.num_programs(0) - 1)
    def _fin():
        LN2 = 0.6931471805599453
        cid = jax.lax.broadcasted_iota(jnp.int32, (1, W), 1)
        col_ok = (cid >= c0) & (cid < c1)
        mp_c = mp_a[...]
        mn_c = jnp.where(col_ok, mn_a[...], NEG)
        sn_c = jnp.where(col_ok, sn_a[...], 0.0)
        Mp = jnp.max(mp_c)
        Mn = jnp.max(mn_c)
        Sp = jnp.sum(sp_a[...] * jnp.exp2(mp_c - Mp))
        Sn = jnp.sum(sn_c * jnp.exp2(mn_c - Mn))
        z = LN2 * (Mp + Mn) + jnp.log(Sp) + jnp.log(Sn)
        o_ref[0, 0] = jnp.maximum(z, 0.0) + jnp.log1p(jnp.exp(-jnp.abs(z)))


def kernel(ypred, truthMask):
    B, H, W = ypred.shape
    mh, mw = truthMask.shape[-2], truthMask.shape[-1]
    r0 = PAD_HT - 1
    r1 = min(PAD_HT - TH + mh + 2, H)
    c0 = PAD_WL - 1
    c1 = min(PAD_WL - TW + mw + 2, W)

    x = ypred.reshape(H, W)
    tm = truthMask.reshape(H, W)

    nrows = 128 if H % 128 == 0 else H
    grid = H // nrows

    out = pl.pallas_call(
        functools.partial(
            _loss_body, nrows=nrows, W=W, r0=r0, r1=r1, c0=c0, c1=c1
        ),
        grid=(grid,),
        in_specs=[
            pl.BlockSpec((nrows, W), lambda i: (i, 0)),
            pl.BlockSpec((nrows, W), lambda i: (i, 0)),
        ],
        out_specs=pl.BlockSpec(
            (1, 1), lambda i: (0, 0), memory_space=pltpu.SMEM
        ),
        out_shape=jax.ShapeDtypeStruct((1, 1), jnp.float32),
        scratch_shapes=[
            pltpu.VMEM((8, W), jnp.float32),
            pltpu.VMEM((8, W), jnp.float32),
            pltpu.VMEM((8, W), jnp.float32),
            pltpu.VMEM((8, W), jnp.float32),
        ],
        compiler_params=pltpu.CompilerParams(
            dimension_semantics=("arbitrary",),
        ),
    )(x, tm)
    return out.reshape(B)


# speedup vs baseline: 1.0424x; 1.0100x over previous
import functools

import jax
import jax.numpy as jnp
from jax.experimental import pallas as pl
from jax.experimental.pallas import tpu as pltpu

TH, TW = 15, 15
PAD_HT = (TH - 1) // 2
PAD_WL = (TW - 1) // 2
MARGIN = 0.25
GAMMA = 256.0
NEG = -1e30
PEN = -1e35


def _loss_body(x_ref, m_ref, o_ref, mp_a, sp_a, mn_a, sn_a, *,
               nrows, W, r0, r1, c0, c1):
    i = pl.program_id(0)
    nsub = nrows // 8

    @pl.when(i == 0)
    def _init():
        mp_a[...] = jnp.full((8, W), NEG, jnp.float32)
        sp_a[...] = jnp.zeros((8, W), jnp.float32)
        mn_a[...] = jnp.full((8, W), NEG, jnp.float32)
        sn_a[...] = jnp.zeros((8, W), jnp.float32)

    x = x_ref[...]
    tm = m_ref[...]

    rid = jax.lax.broadcasted_iota(jnp.int32, (nrows, 1), 0) + i * nrows
    rowpen = jnp.where((rid >= r0) & (rid < r1), 0.0, PEN)

    K = GAMMA * 1.4426950408889634
    y = x * K
    lp = jnp.maximum(K * (1.0 + MARGIN) - y, 0.0) * ((1.0 - MARGIN) - x)
    ln = jnp.maximum(y + K * MARGIN, 0.0) * (x - MARGIN) + rowpen
    lP = jnp.where(tm, lp, PEN).reshape(nsub, 8, W)
    lN = jnp.where(tm, PEN, ln).reshape(nsub, 8, W)

    mp_old = mp_a[...]
    mn_old = mn_a[...]
    mp = jnp.maximum(mp_old, jnp.max(lP, axis=0))
    mn = jnp.maximum(mn_old, jnp.max(lN, axis=0))

    ep = jnp.exp2(lP - mp[None, :, :])
    en = jnp.exp2(lN - mn[None, :, :])

    mp_a[...] = mp
    sp_a[...] = sp_a[...] * jnp.exp2(mp_old - mp) + jnp.sum(ep, axis=0)
    mn_a[...] = mn
    sn_a[...] = sn_a[...] * jnp.exp2(mn_old - mn) + jnp.sum(en, axis=0)

    @pl.when(i == pl.num_programs(0) - 1)
    def _fin():
        LN2 = 0.6931471805599453
        cid = jax.lax.broadcasted_iota(jnp.int32, (1, W), 1)
        col_ok = (cid >= c0) & (cid < c1)
        mp_c = mp_a[...]
        mn_c = jnp.where(col_ok, mn_a[...], NEG)
        sn_c = jnp.where(col_ok, sn_a[...], 0.0)
        Mp = jnp.max(mp_c)
        Mn = jnp.max(mn_c)
        Sp = jnp.sum(sp_a[...] * jnp.exp2(mp_c - Mp))
        Sn = jnp.sum(sn_c * jnp.exp2(mn_c - Mn))
        z = LN2 * (Mp + Mn) + jnp.log(Sp) + jnp.log(Sn)
        o_ref[0, 0] = jnp.maximum(z, 0.0) + jnp.log1p(jnp.exp(-jnp.abs(z)))


def kernel(ypred, truthMask):
    B, H, W = ypred.shape
    mh, mw = truthMask.shape[-2], truthMask.shape[-1]
    r0 = PAD_HT - 1
    r1 = min(PAD_HT - TH + mh + 2, H)
    c0 = PAD_WL - 1
    c1 = min(PAD_WL - TW + mw + 2, W)

    x = ypred.reshape(H, W)
    tm = truthMask.reshape(H, W)

    nrows = 256 if H % 256 == 0 else H
    grid = H // nrows

    out = pl.pallas_call(
        functools.partial(
            _loss_body, nrows=nrows, W=W, r0=r0, r1=r1, c0=c0, c1=c1
        ),
        grid=(grid,),
        in_specs=[
            pl.BlockSpec((nrows, W), lambda i: (i, 0)),
            pl.BlockSpec((nrows, W), lambda i: (i, 0)),
        ],
        out_specs=pl.BlockSpec(
            (1, 1), lambda i: (0, 0), memory_space=pltpu.SMEM
        ),
        out_shape=jax.ShapeDtypeStruct((1, 1), jnp.float32),
        scratch_shapes=[
            pltpu.VMEM((8, W), jnp.float32),
            pltpu.VMEM((8, W), jnp.float32),
            pltpu.VMEM((8, W), jnp.float32),
            pltpu.VMEM((8, W), jnp.float32),
        ],
        compiler_params=pltpu.CompilerParams(
            dimension_semantics=("arbitrary",),
        ),
    )(x, tm)
    return out.reshape(B)
